# baseline (speedup 1.0000x reference)
import functools

import jax
import jax.numpy as jnp
from jax import lax
from jax.experimental import pallas as pl
from jax.experimental.pallas import tpu as pltpu
from jax.experimental.pallas import tpu_sc as plsc

_NC = 2
_NS = 16
_L = 16
_NW = _NC * _NS

_CHUNK = 80
_DW = 16


def _sc_aggregate(src, dst, e_id, edge_weight, x, znd, zdeg):
    e_total = src.shape[0]
    n, d = x.shape
    ew_per_worker = e_total // _NW
    n_chunks = ew_per_worker // _CHUNK
    mesh = plsc.VectorSubcoreMesh(core_axis_name="c", subcore_axis_name="s")

    @functools.partial(
        pl.kernel,
        out_type=(
            jax.ShapeDtypeStruct((_NC, n, d), jnp.float32),
            jax.ShapeDtypeStruct((_NC, n, _DW), jnp.float32),
        ),
        mesh=mesh,
        compiler_params=pltpu.CompilerParams(
            use_tc_tiling_on_sc=False, needs_layout_passes=False),
        scratch_types=[
            [pltpu.VMEM((_CHUNK,), jnp.int32)] * 2,
            [pltpu.VMEM((_CHUNK,), jnp.int32)] * 2,
            [pltpu.VMEM((_CHUNK,), jnp.int32)] * 2,
            [pltpu.VMEM((_CHUNK,), jnp.int32)] * 2,
            [pltpu.VMEM((_CHUNK,), jnp.float32)] * 2,
            [pltpu.VMEM((_CHUNK, d), jnp.bfloat16)] * 2,
            [pltpu.VMEM((_CHUNK, d), jnp.float32)] * 2,
            pltpu.VMEM((_CHUNK, _DW), jnp.float32),
            pltpu.VMEM_SHARED((n, d), jnp.float32),
            pltpu.VMEM_SHARED((n, _DW), jnp.float32),
            [pltpu.SemaphoreType.DMA] * 2,
            [pltpu.SemaphoreType.DMA] * 2,
            [pltpu.SemaphoreType.DMA] * 2,
            [pltpu.SemaphoreType.DMA] * 2,
            [pltpu.SemaphoreType.DMA] * 2,
        ],
    )
    def sc_kernel(src_h, dst_h, eid_h, w_h, x_h, znd_h, zdeg_h,
                  acc_out_h, deg_out_h,
                  srcb, dstb, eidb, dsts, wb, rows16, rowsb, ones_v,
                  acc_s, deg_s, si, sw, sx, ss, so):
        cid = lax.axis_index("c")
        sid = lax.axis_index("s")
        wid = sid * _NC + cid
        base = wid * ew_per_worker
        one_row = jnp.full((_L,), 1.0, dtype=jnp.float32)

        @pl.when(sid == 0)
        def _init():
            pltpu.sync_copy(znd_h, acc_s)
            pltpu.sync_copy(zdeg_h, deg_s)

        def _fill_ones(i, c):
            ones_v[i, :] = one_row
            return c

        lax.fori_loop(0, _CHUNK, _fill_ones, 0)
        plsc.subcore_barrier()

        def _off(t):
            return pl.multiple_of(base + t * _CHUNK, 8)

        def issue_idx(t, k):
            off = _off(t)
            pltpu.async_copy(src_h.at[pl.ds(off, _CHUNK)], srcb[k], si[k])
            pltpu.async_copy(dst_h.at[pl.ds(off, _CHUNK)], dstb[k], si[k])
            pltpu.async_copy(eid_h.at[pl.ds(off, _CHUNK)], eidb[k], si[k])

        def wait_idx(t, k):
            off = _off(t)
            pltpu.make_async_copy(src_h.at[pl.ds(off, _CHUNK)], srcb[k], si[k]).wait()
            pltpu.make_async_copy(dst_h.at[pl.ds(off, _CHUNK)], dstb[k], si[k]).wait()
            pltpu.make_async_copy(eid_h.at[pl.ds(off, _CHUNK)], eidb[k], si[k]).wait()

        def issue_gathers(k):
            pltpu.async_copy(w_h.at[eidb[k]], wb[k], sw[k])
            pltpu.async_copy(x_h.at[srcb[k]], rows16[k], sx[k])

        def wait_gathers(k):
            pltpu.make_async_copy(w_h.at[eidb[k]], wb[k], sw[k]).wait()
            pltpu.make_async_copy(x_h.at[srcb[k]], rows16[k], sx[k]).wait()

        def copy_dst(k):
            for i in range(_CHUNK // _L):
                sl = pl.ds(i * _L, _L)
                dsts[k][sl] = dstb[k][sl]

        def scale(k):
            def _scale(g, c):
                e0 = g * _L
                wv = wb[k][pl.ds(e0, _L)]
                for q in range(_L):
                    w = wv[q]
                    for j in range(d // (2 * _L)):
                        b32 = rows16[k][e0 + q, pl.ds(j * 2 * _L, 2 * _L)]
                        lo, hi = plsc.unpack(b32, format=plsc.PackFormat.INTERLEAVED)
                        rowsb[k][e0 + q, pl.ds(j * 2 * _L, _L)] = lo * w
                        rowsb[k][e0 + q, pl.ds(j * 2 * _L + _L, _L)] = hi * w
                return c

            lax.fori_loop(0, _CHUNK // _L, _scale, 0)

        def issue_scatters(k):
            pltpu.async_copy(rowsb[k], acc_s.at[dsts[k]], ss[k], add=True)
            pltpu.async_copy(ones_v, deg_s.at[dsts[k]], so[k], add=True)

        def wait_scatters(k):
            pltpu.make_async_copy(rowsb[k], acc_s.at[dsts[k]], ss[k]).wait()
            pltpu.make_async_copy(ones_v, deg_s.at[dsts[k]], so[k]).wait()

        def _sub_body(t, k):
            b = 1 - k

            @pl.when(t < n_chunks)
            def _():
                wait_gathers(k)
                copy_dst(k)

                @pl.when(t + 1 < n_chunks)
                def _():
                    wait_idx(t + 1, b)

                @pl.when(t >= 1)
                def _():
                    wait_scatters(b)

                @pl.when(t + 1 < n_chunks)
                def _():
                    issue_gathers(b)

                @pl.when(t + 2 < n_chunks)
                def _():
                    issue_idx(t + 2, k)

                scale(k)
                issue_scatters(k)

        issue_idx(0, 0)
        wait_idx(0, 0)
        issue_gathers(0)
        issue_idx(1, 1)

        def _pair(g, c):
            _sub_body(2 * g, 0)
            _sub_body(2 * g + 1, 1)
            return c

        lax.fori_loop(0, (n_chunks + 1) // 2, _pair, 0)
        wait_scatters((n_chunks - 1) % 2)
        plsc.subcore_barrier()

        @pl.when(sid == 0)
        def _flush():
            pltpu.sync_copy(acc_s, acc_out_h.at[cid])
            pltpu.sync_copy(deg_s, deg_out_h.at[cid])

    return sc_kernel(src, dst, e_id, edge_weight, x, znd, zdeg)


def _tc_body(p_ref, deg_ref, x_ref, wr_ref, br_ref, wo_ref, out_ref, ls_ref):
    p = p_ref[0] + p_ref[1]
    deg = deg_ref[0, :, 0:1] + deg_ref[1, :, 0:1]
    agg = p / jnp.maximum(deg, 1.0)
    out = (
        lax.dot_general(agg, wr_ref[...], (((1,), (1,)), ((), ())),
                        preferred_element_type=jnp.float32)
        + br_ref[...]
        + lax.dot_general(x_ref[...], wo_ref[...], (((1,), (1,)), ((), ())),
                          preferred_element_type=jnp.float32)
    )
    out_ref[...] = out
    m = jnp.max(out, axis=1, keepdims=True)
    s = out - m
    ls_ref[...] = s - jnp.log(jnp.sum(jnp.exp(s), axis=1, keepdims=True))


def _tc_finish(parts, degs, x, w_rel, b_rel, w_root):
    n, d = x.shape
    r = 1000
    grid = (n // r,)
    return pl.pallas_call(
        _tc_body,
        grid=grid,
        in_specs=[
            pl.BlockSpec((_NC, r, d), lambda i: (0, i, 0)),
            pl.BlockSpec((_NC, r, _DW), lambda i: (0, i, 0)),
            pl.BlockSpec((r, d), lambda i: (i, 0)),
            pl.BlockSpec((d, d), lambda i: (0, 0)),
            pl.BlockSpec((1, d), lambda i: (0, 0)),
            pl.BlockSpec((d, d), lambda i: (0, 0)),
        ],
        out_specs=[
            pl.BlockSpec((r, d), lambda i: (i, 0)),
            pl.BlockSpec((r, d), lambda i: (i, 0)),
        ],
        out_shape=[
            jax.ShapeDtypeStruct((n, d), jnp.float32),
            jax.ShapeDtypeStruct((n, d), jnp.float32),
        ],
        compiler_params=pltpu.CompilerParams(
            dimension_semantics=("parallel",),
        ),
    )(parts, degs, x, w_rel, b_rel.reshape(1, d), w_root)


def kernel(x, edge_index, e_id, edge_weight, size_dst, W_rel, b_rel, W_root):
    n, d = x.shape
    src = edge_index[0]
    dst = edge_index[1]
    znd = jnp.zeros((n, d), dtype=jnp.float32)
    zdeg = jnp.zeros((n, _DW), dtype=jnp.float32)
    cols = jnp.arange(d).reshape(d // 32, 2, 16).transpose(0, 2, 1).reshape(d)
    xb = x.astype(jnp.bfloat16)[:, cols]
    parts, degs = _sc_aggregate(src, dst, e_id, edge_weight, xb, znd, zdeg)
    out, logsm = _tc_finish(parts, degs, x, W_rel, b_rel, W_root)
    return (out, logsm)


# speedup vs baseline: 1.7406x; 1.7406x over previous
import functools

import jax
import jax.numpy as jnp
from jax import lax
from jax.experimental import pallas as pl
from jax.experimental.pallas import tpu as pltpu
from jax.experimental.pallas import tpu_sc as plsc

_NC = 2
_NS = 16
_L = 16
_NW = _NC * _NS

_CHUNK = 80
_DW = 16


def _sc_aggregate(src, dst, e_id, edge_weight, x, znd, zdeg):
    e_total = src.shape[0]
    n, d = x.shape
    ew_per_worker = e_total // _NW
    n_chunks = ew_per_worker // _CHUNK
    rows_per_tile = n // _NS
    mesh = plsc.VectorSubcoreMesh(core_axis_name="c", subcore_axis_name="s")

    @functools.partial(
        pl.kernel,
        out_type=(
            jax.ShapeDtypeStruct((_NC, n, d), jnp.float32),
            jax.ShapeDtypeStruct((_NC, n, _DW), jnp.float32),
        ),
        mesh=mesh,
        compiler_params=pltpu.CompilerParams(use_tc_tiling_on_sc=False),
        scratch_types=[
            [pltpu.VMEM((_CHUNK,), jnp.int32)] * 2,
            [pltpu.VMEM((_CHUNK,), jnp.int32)] * 2,
            [pltpu.VMEM((_CHUNK,), jnp.int32)] * 2,
            [pltpu.VMEM((_CHUNK,), jnp.int32)] * 2,
            [pltpu.VMEM((_CHUNK,), jnp.float32)] * 2,
            [pltpu.VMEM((_CHUNK, d), jnp.float32)] * 2,
            pltpu.VMEM((_CHUNK, _DW), jnp.float32),
            pltpu.VMEM_SHARED((n, d), jnp.float32),
            pltpu.VMEM_SHARED((n, _DW), jnp.float32),
            [pltpu.SemaphoreType.DMA] * 2,
            [pltpu.SemaphoreType.DMA] * 2,
            [pltpu.SemaphoreType.DMA] * 2,
            [pltpu.SemaphoreType.DMA] * 2,
            [pltpu.SemaphoreType.DMA] * 2,
        ],
    )
    def sc_kernel(src_h, dst_h, eid_h, w_h, x_h, znd_h, zdeg_h,
                  acc_out_h, deg_out_h,
                  srcb, dstb, eidb, dsts, wb, rowsb, ones_v,
                  acc_s, deg_s, si, sw, sx, ss, so):
        cid = lax.axis_index("c")
        sid = lax.axis_index("s")
        wid = sid * _NC + cid
        base = wid * ew_per_worker
        row0 = sid * rows_per_tile
        one_row = jnp.full((_L,), 1.0, dtype=jnp.float32)
        rsl = pl.ds(pl.multiple_of(row0, 8), rows_per_tile)

        def _off(t):
            return pl.multiple_of(base + t * _CHUNK, 8)

        def issue_idx(t, k):
            off = _off(t)
            pltpu.async_copy(src_h.at[pl.ds(off, _CHUNK)], srcb[k], si[k])
            pltpu.async_copy(dst_h.at[pl.ds(off, _CHUNK)], dstb[k], si[k])
            pltpu.async_copy(eid_h.at[pl.ds(off, _CHUNK)], eidb[k], si[k])

        def wait_idx(t, k):
            off = _off(t)
            pltpu.make_async_copy(src_h.at[pl.ds(off, _CHUNK)], srcb[k], si[k]).wait()
            pltpu.make_async_copy(dst_h.at[pl.ds(off, _CHUNK)], dstb[k], si[k]).wait()
            pltpu.make_async_copy(eid_h.at[pl.ds(off, _CHUNK)], eidb[k], si[k]).wait()

        def issue_gathers(k):
            pltpu.async_copy(w_h.at[eidb[k]], wb[k], sw[k])
            pltpu.async_copy(x_h.at[srcb[k]], rowsb[k], sx[k])

        def wait_gathers(k):
            pltpu.make_async_copy(w_h.at[eidb[k]], wb[k], sw[k]).wait()
            pltpu.make_async_copy(x_h.at[srcb[k]], rowsb[k], sx[k]).wait()

        def copy_dst(k):
            for i in range(_CHUNK // _L):
                sl = pl.ds(i * _L, _L)
                dsts[k][sl] = dstb[k][sl]

        def scale(k):
            def _scale(g, c):
                e0 = g * _L
                wv = wb[k][pl.ds(e0, _L)]
                for q in range(_L):
                    w = wv[q]
                    for j in range(d // _L):
                        sl = pl.ds(j * _L, _L)
                        rowsb[k][e0 + q, sl] = rowsb[k][e0 + q, sl] * w
                return c

            lax.fori_loop(0, _CHUNK // _L, _scale, 0)

        def issue_scatters(k):
            pltpu.async_copy(rowsb[k], acc_s.at[dsts[k]], ss[k], add=True)
            pltpu.async_copy(ones_v, deg_s.at[dsts[k]], so[k], add=True)

        def wait_scatters(k):
            pltpu.make_async_copy(rowsb[k], acc_s.at[dsts[k]], ss[k]).wait()
            pltpu.make_async_copy(ones_v, deg_s.at[dsts[k]], so[k]).wait()

        issue_idx(0, 0)
        issue_idx(1, 1)
        pltpu.sync_copy(znd_h.at[rsl], acc_s.at[rsl])
        pltpu.sync_copy(zdeg_h.at[rsl], deg_s.at[rsl])

        def _fill_ones(i, c):
            ones_v[i, :] = one_row
            return c

        lax.fori_loop(0, _CHUNK, _fill_ones, 0)

        wait_idx(0, 0)
        issue_gathers(0)
        plsc.subcore_barrier()

        def _sub_body(t, k):
            b = 1 - k

            @pl.when(t < n_chunks)
            def _():
                wait_gathers(k)
                copy_dst(k)

                @pl.when(t + 1 < n_chunks)
                def _():
                    wait_idx(t + 1, b)

                @pl.when(t >= 1)
                def _():
                    wait_scatters(b)

                @pl.when(t + 1 < n_chunks)
                def _():
                    issue_gathers(b)

                @pl.when(t + 2 < n_chunks)
                def _():
                    issue_idx(t + 2, k)

                scale(k)
                issue_scatters(k)

        def _pair(g, c):
            _sub_body(2 * g, 0)
            _sub_body(2 * g + 1, 1)
            return c

        lax.fori_loop(0, (n_chunks + 1) // 2, _pair, 0)
        wait_scatters((n_chunks - 1) % 2)
        plsc.subcore_barrier()

        pltpu.sync_copy(acc_s.at[rsl], acc_out_h.at[cid, rsl])
        pltpu.sync_copy(deg_s.at[rsl], deg_out_h.at[cid, rsl])

    return sc_kernel(src, dst, e_id, edge_weight, x, znd, zdeg)


def _tc_body(p_ref, deg_ref, x_ref, wr_ref, br_ref, wo_ref, out_ref, ls_ref):
    p = p_ref[0] + p_ref[1]
    deg = deg_ref[0, :, 0:1] + deg_ref[1, :, 0:1]
    agg = p / jnp.maximum(deg, 1.0)
    out = (
        lax.dot_general(agg, wr_ref[...], (((1,), (1,)), ((), ())),
                        preferred_element_type=jnp.float32)
        + br_ref[...]
        + lax.dot_general(x_ref[...], wo_ref[...], (((1,), (1,)), ((), ())),
                          preferred_element_type=jnp.float32)
    )
    out_ref[...] = out
    m = jnp.max(out, axis=1, keepdims=True)
    s = out - m
    ls_ref[...] = s - jnp.log(jnp.sum(jnp.exp(s), axis=1, keepdims=True))


def _tc_finish(parts, degs, x, w_rel, b_rel, w_root):
    n, d = x.shape
    r = 1000
    grid = (n // r,)
    return pl.pallas_call(
        _tc_body,
        grid=grid,
        in_specs=[
            pl.BlockSpec((_NC, r, d), lambda i: (0, i, 0)),
            pl.BlockSpec((_NC, r, _DW), lambda i: (0, i, 0)),
            pl.BlockSpec((r, d), lambda i: (i, 0)),
            pl.BlockSpec((d, d), lambda i: (0, 0)),
            pl.BlockSpec((1, d), lambda i: (0, 0)),
            pl.BlockSpec((d, d), lambda i: (0, 0)),
        ],
        out_specs=[
            pl.BlockSpec((r, d), lambda i: (i, 0)),
            pl.BlockSpec((r, d), lambda i: (i, 0)),
        ],
        out_shape=[
            jax.ShapeDtypeStruct((n, d), jnp.float32),
            jax.ShapeDtypeStruct((n, d), jnp.float32),
        ],
        compiler_params=pltpu.CompilerParams(
            dimension_semantics=("parallel",),
        ),
    )(parts, degs, x, w_rel, b_rel.reshape(1, d), w_root)


def kernel(x, edge_index, e_id, edge_weight, size_dst, W_rel, b_rel, W_root):
    n, d = x.shape
    src = edge_index[0]
    dst = edge_index[1]
    znd = jnp.zeros((n, d), dtype=jnp.float32)
    zdeg = jnp.zeros((n, _DW), dtype=jnp.float32)
    parts, degs = _sc_aggregate(src, dst, e_id, edge_weight, x, znd, zdeg)
    out, logsm = _tc_finish(parts, degs, x, W_rel, b_rel, W_root)
    return (out, logsm)
